# baseline (speedup 1.0000x reference)
import jax
import jax.numpy as jnp
from jax.experimental import pallas as pl
from jax.experimental.pallas import tpu as pltpu

_N_CHUNKS = 5
_CHUNK = 2000


def _hgnn_kernel(x_hbm, H_hbm, W1_ref, W2_ref, b1_ref, b2_ref, Wc_ref,
                 bc_ref, out_ref, xv_ref, Hs_ref, Hf_ref, xw_ref, sems):
    f32 = jnp.float32
    nc, c = _N_CHUNKS, _CHUNK

    for k in range(nc):
        sl = pl.ds(k * c, c)
        pltpu.make_async_copy(x_hbm.at[sl, :], xv_ref.at[sl, :],
                              sems.at[k]).start()
        pltpu.make_async_copy(H_hbm.at[sl, :], Hs_ref.at[sl, :],
                              sems.at[nc + k]).start()

    acc = jnp.zeros((H_hbm.shape[1], W1_ref.shape[1]), f32)
    bdeg = jnp.zeros((H_hbm.shape[1], 1), f32)
    ones = jnp.ones((c, 1), f32)
    for k in range(nc):
        sl = pl.ds(k * c, c)
        pltpu.make_async_copy(x_hbm.at[sl, :], xv_ref.at[sl, :],
                              sems.at[k]).wait()
        xw_k = jnp.dot(xv_ref[sl, :], W1_ref[...], preferred_element_type=f32)
        xw_ref[sl, :] = xw_k
        pltpu.make_async_copy(H_hbm.at[sl, :], Hs_ref.at[sl, :],
                              sems.at[nc + k]).wait()
        Hf_k = Hs_ref[sl, :].astype(f32)
        Hf_ref[sl, :] = Hf_k
        acc += jax.lax.dot_general(
            Hf_k, xw_k, (((0,), (0,)), ((), ())), preferred_element_type=f32)
        bdeg += jax.lax.dot_general(
            Hf_k, ones, (((0,), (0,)), ((), ())), preferred_element_type=f32)

    binv = jnp.where(bdeg > 0, 1.0 / bdeg, 0.0)
    Hf = Hf_ref[...]
    deg = jnp.sum(Hf, axis=1, keepdims=True)
    dinv = jnp.where(deg > 0, 1.0 / deg, 0.0)

    m = binv * acc
    h = jax.nn.relu(
        dinv * jnp.dot(Hf, m, preferred_element_type=f32) + b1_ref[...])
    hw = jnp.dot(h, W2_ref[...], preferred_element_type=f32)
    m2 = binv * jax.lax.dot_general(
        Hf, hw, (((0,), (0,)), ((), ())), preferred_element_type=f32)
    h2 = jax.nn.relu(
        dinv * jnp.dot(Hf, m2, preferred_element_type=f32) + b2_ref[...])

    out_ref[...] = (
        jnp.dot(h2, Wc_ref[...], preferred_element_type=f32) + bc_ref[...])


def kernel(x, H, edge_weights, W1, b1, W2, b2, Wc, bc):
    del edge_weights
    n, d_in = x.shape
    e_h = H.shape[1]
    d_hid = W1.shape[1]

    out = pl.pallas_call(
        _hgnn_kernel,
        in_specs=[
            pl.BlockSpec(memory_space=pl.ANY),
            pl.BlockSpec(memory_space=pl.ANY),
            pl.BlockSpec((d_in, d_hid), lambda: (0, 0)),
            pl.BlockSpec((d_hid, d_hid), lambda: (0, 0)),
            pl.BlockSpec((1, d_hid), lambda: (0, 0)),
            pl.BlockSpec((1, d_hid), lambda: (0, 0)),
            pl.BlockSpec((d_hid, 1), lambda: (0, 0)),
            pl.BlockSpec((1, 1), lambda: (0, 0)),
        ],
        out_shape=jax.ShapeDtypeStruct((n, 1), jnp.float32),
        scratch_shapes=[
            pltpu.VMEM((n, d_in), jnp.float32),
            pltpu.VMEM((n, e_h), jnp.int32),
            pltpu.VMEM((n, e_h), jnp.float32),
            pltpu.VMEM((n, d_hid), jnp.float32),
            pltpu.SemaphoreType.DMA((2 * _N_CHUNKS,)),
        ],
    )(x, H, W1, W2, b1.reshape(1, d_hid), b2.reshape(1, d_hid), Wc,
      bc.reshape(1, 1))

    return out


# speedup vs baseline: 1.0092x; 1.0092x over previous
import jax
import jax.numpy as jnp
from jax.experimental import pallas as pl
from jax.experimental.pallas import tpu as pltpu

_N_CHUNKS = 5
_CHUNK = 2000


def _hgnn_kernel(x_hbm, H_hbm, W1_ref, W2_ref, b1_ref, b2_ref, Wc_ref,
                 bc_ref, out_ref, xv_ref, Hs_ref, sems):
    f32 = jnp.float32
    nc, c = _N_CHUNKS, _CHUNK

    for k in range(nc):
        sl = pl.ds(k * c, c)
        pltpu.make_async_copy(x_hbm.at[sl, :], xv_ref.at[sl, :],
                              sems.at[k]).start()
        pltpu.make_async_copy(H_hbm.at[sl, :], Hs_ref.at[sl, :],
                              sems.at[nc + k]).start()

    acc = jnp.zeros((H_hbm.shape[1], W1_ref.shape[1]), f32)
    bdeg = jnp.zeros((H_hbm.shape[1], 1), f32)
    ones = jnp.ones((c, 1), f32)
    for k in range(nc):
        sl = pl.ds(k * c, c)
        pltpu.make_async_copy(x_hbm.at[sl, :], xv_ref.at[sl, :],
                              sems.at[k]).wait()
        xw_k = jnp.dot(xv_ref[sl, :], W1_ref[...], preferred_element_type=f32)
        pltpu.make_async_copy(H_hbm.at[sl, :], Hs_ref.at[sl, :],
                              sems.at[nc + k]).wait()
        Hf_k = Hs_ref[sl, :].astype(f32)
        acc += jax.lax.dot_general(
            Hf_k, xw_k, (((0,), (0,)), ((), ())), preferred_element_type=f32)
        bdeg += jax.lax.dot_general(
            Hf_k, ones, (((0,), (0,)), ((), ())), preferred_element_type=f32)

    binv = jnp.where(bdeg > 0, 1.0 / bdeg, 0.0)
    Hf = Hs_ref[...].astype(f32)
    deg = jnp.sum(Hf, axis=1, keepdims=True)
    dinv = jnp.where(deg > 0, 1.0 / deg, 0.0)

    m = binv * acc
    h = jax.nn.relu(
        dinv * jnp.dot(Hf, m, preferred_element_type=f32) + b1_ref[...])
    hw = jnp.dot(h, W2_ref[...], preferred_element_type=f32)
    m2 = binv * jax.lax.dot_general(
        Hf, hw, (((0,), (0,)), ((), ())), preferred_element_type=f32)
    h2 = jax.nn.relu(
        dinv * jnp.dot(Hf, m2, preferred_element_type=f32) + b2_ref[...])

    out_ref[...] = (
        jnp.dot(h2, Wc_ref[...], preferred_element_type=f32) + bc_ref[...])


def kernel(x, H, edge_weights, W1, b1, W2, b2, Wc, bc):
    del edge_weights
    n, d_in = x.shape
    e_h = H.shape[1]
    d_hid = W1.shape[1]

    out = pl.pallas_call(
        _hgnn_kernel,
        in_specs=[
            pl.BlockSpec(memory_space=pl.ANY),
            pl.BlockSpec(memory_space=pl.ANY),
            pl.BlockSpec((d_in, d_hid), lambda: (0, 0)),
            pl.BlockSpec((d_hid, d_hid), lambda: (0, 0)),
            pl.BlockSpec((1, d_hid), lambda: (0, 0)),
            pl.BlockSpec((1, d_hid), lambda: (0, 0)),
            pl.BlockSpec((d_hid, 1), lambda: (0, 0)),
            pl.BlockSpec((1, 1), lambda: (0, 0)),
        ],
        out_shape=jax.ShapeDtypeStruct((n, 1), jnp.float32),
        scratch_shapes=[
            pltpu.VMEM((n, d_in), jnp.float32),
            pltpu.VMEM((n, e_h), jnp.int32),
            pltpu.SemaphoreType.DMA((2 * _N_CHUNKS,)),
        ],
    )(x, H, W1, W2, b1.reshape(1, d_hid), b2.reshape(1, d_hid), Wc,
      bc.reshape(1, 1))

    return out


# speedup vs baseline: 1.0368x; 1.0273x over previous
import jax
import jax.numpy as jnp
from jax.experimental import pallas as pl
from jax.experimental.pallas import tpu as pltpu


def _hgnn_kernel(x_ref, H_ref, W1_ref, W2_ref, b1_ref, b2_ref, Wc_ref,
                 bc_ref, out_ref):
    f32 = jnp.float32
    Hf = H_ref[...].astype(f32)
    ones = jnp.ones((Hf.shape[0], 1), f32)
    bdeg = jax.lax.dot_general(
        Hf, ones, (((0,), (0,)), ((), ())), preferred_element_type=f32)
    binv = jnp.where(bdeg > 0, 1.0 / bdeg, 0.0)
    deg = jnp.sum(Hf, axis=1, keepdims=True)
    dinv = jnp.where(deg > 0, 1.0 / deg, 0.0)

    xw = jnp.dot(x_ref[...], W1_ref[...], preferred_element_type=f32)
    m = binv * jax.lax.dot_general(
        Hf, xw, (((0,), (0,)), ((), ())), preferred_element_type=f32)
    h = jax.nn.relu(
        dinv * jnp.dot(Hf, m, preferred_element_type=f32) + b1_ref[...])

    hw = jnp.dot(h, W2_ref[...], preferred_element_type=f32)
    m2 = binv * jax.lax.dot_general(
        Hf, hw, (((0,), (0,)), ((), ())), preferred_element_type=f32)
    h2 = jax.nn.relu(
        dinv * jnp.dot(Hf, m2, preferred_element_type=f32) + b2_ref[...])

    out_ref[...] = (
        jnp.dot(h2, Wc_ref[...], preferred_element_type=f32) + bc_ref[...])


def kernel(x, H, edge_weights, W1, b1, W2, b2, Wc, bc):
    del edge_weights
    n, d_in = x.shape
    d_hid = W1.shape[1]

    out = pl.pallas_call(
        _hgnn_kernel,
        out_shape=jax.ShapeDtypeStruct((n, 1), jnp.float32),
    )(x, H, W1, W2, b1.reshape(1, d_hid), b2.reshape(1, d_hid), Wc,
      bc.reshape(1, 1))

    return out


# speedup vs baseline: 1.0652x; 1.0274x over previous
import jax
import jax.numpy as jnp
from jax.experimental import pallas as pl
from jax.experimental.pallas import tpu as pltpu


def _hgnn_kernel(x_ref, H_ref, W1_ref, W2_ref, b1_ref, b2_ref, Wc_ref,
                 bc_ref, out_ref):
    f32 = jnp.float32
    Hf = H_ref[...].astype(f32)
    ones = jnp.ones((Hf.shape[0], 1), f32)
    bdeg = jax.lax.dot_general(
        Hf, ones, (((0,), (0,)), ((), ())), preferred_element_type=f32)
    binv = jnp.where(bdeg > 0, 1.0 / bdeg, 0.0)
    deg = jnp.sum(Hf, axis=1, keepdims=True)
    dinv = jnp.where(deg > 0, 1.0 / deg, 0.0)

    t1 = jax.lax.dot_general(
        Hf, x_ref[...], (((0,), (0,)), ((), ())), preferred_element_type=f32)
    m = binv * jnp.dot(t1, W1_ref[...], preferred_element_type=f32)
    h = jax.nn.relu(
        dinv * jnp.dot(Hf, m, preferred_element_type=f32) + b1_ref[...])

    t2 = jax.lax.dot_general(
        Hf, h, (((0,), (0,)), ((), ())), preferred_element_type=f32)
    m2 = binv * jnp.dot(t2, W2_ref[...], preferred_element_type=f32)
    h2 = jax.nn.relu(
        dinv * jnp.dot(Hf, m2, preferred_element_type=f32) + b2_ref[...])

    out_ref[...] = (
        jnp.dot(h2, Wc_ref[...], preferred_element_type=f32) + bc_ref[...])


def kernel(x, H, edge_weights, W1, b1, W2, b2, Wc, bc):
    del edge_weights
    n, d_in = x.shape
    d_hid = W1.shape[1]

    out = pl.pallas_call(
        _hgnn_kernel,
        out_shape=jax.ShapeDtypeStruct((n, 1), jnp.float32),
    )(x, H, W1, W2, b1.reshape(1, d_hid), b2.reshape(1, d_hid), Wc,
      bc.reshape(1, 1))

    return out
